# baseline (speedup 1.0000x reference)
import functools

import jax
import jax.numpy as jnp
from jax import lax
from jax.experimental import pallas as pl
from jax.experimental.pallas import tpu as pltpu
from jax.experimental.pallas import tpu_sc as plsc

B = 1024
L = 50
D = 64
V = 119547

NC = 2
NS = 16
NW = NC * NS
BPW = B // NW

LANES = 16


CT = 2048
NPREP = (V + CT - 1) // CT
WAVE = BPW // 2


def _tc_prep_table(wt_table):
    def body(t_ref, o_ref):
        eye = (lax.broadcasted_iota(jnp.int32, (D, D), 0)
               == lax.broadcasted_iota(jnp.int32, (D, D), 1)
               ).astype(jnp.float32)
        o_ref[:, 0:D] = lax.dot_general(
            t_ref[...], eye, (((0,), (0,)), ((), ())),
            preferred_element_type=jnp.float32)

    return pl.pallas_call(
        body,
        grid=(NPREP,),
        in_specs=[pl.BlockSpec((D, CT), lambda i: (0, i))],
        out_specs=pl.BlockSpec((CT, 128), lambda i: (i, 0)),
        out_shape=jax.ShapeDtypeStruct((V, 128), jnp.float32),
    )(wt_table)


def _sc_embed_mean(x, table128):
    mesh = plsc.VectorSubcoreMesh(core_axis_name="c", subcore_axis_name="s")

    @functools.partial(
        pl.kernel,
        out_type=jax.ShapeDtypeStruct((B, D), jnp.float32),
        mesh=mesh,
        compiler_params=pltpu.CompilerParams(use_tc_tiling_on_sc=False),
        scratch_types=[
            pltpu.VMEM((BPW, L), jnp.int32),
            pltpu.VMEM((WAVE * L, 128), jnp.float32),
            pltpu.VMEM((BPW, D), jnp.float32),
            pltpu.SemaphoreType.DMA,
        ],
    )
    def k(x_hbm, table_hbm, out_hbm, idx_v, rows_v, h_v, sem):
        wid = lax.axis_index("s") * NC + lax.axis_index("c")
        base = wid * BPW
        pltpu.sync_copy(x_hbm.at[pl.ds(base, BPW)], idx_v)

        @pl.loop(0, BPW, step=WAVE)
        def _(w):
            @pl.loop(0, WAVE)
            def _(j):
                pltpu.async_copy(
                    table_hbm.at[idx_v.at[w + j]],
                    rows_v.at[pl.ds(j * L, L)], sem)

            @pl.loop(0, WAVE)
            def _(j):
                pltpu.make_async_copy(
                    table_hbm.at[idx_v.at[w + j]],
                    rows_v.at[pl.ds(j * L, L)], sem
                ).wait()

            @pl.loop(0, WAVE)
            def _(j):
                for c in range(0, D, LANES):
                    acc = rows_v[j * L, pl.ds(c, LANES)]
                    for l in range(1, L):
                        acc = acc + rows_v[j * L + l, pl.ds(c, LANES)]
                    h_v[w + j, pl.ds(c, LANES)] = acc * (1.0 / L)

        pltpu.sync_copy(h_v, out_hbm.at[pl.ds(base, BPW)])

    return k(x, table128)


VT = 4096
GRID_V = (V + VT - 1) // VT


NFULL = V // VT
REM = V - NFULL * VT


def _tc_logits(h, W, b2):
    def body(h_ref, w_ref, b_ref, o_hbm, acc, sems):
        i = pl.program_id(0)
        slot = lax.rem(i, 2)

        for s in (0, 1):
            @pl.when(slot == s)
            def _(s=s):
                @pl.when(i >= 2)
                def _():
                    pltpu.make_async_copy(
                        acc.at[s],
                        o_hbm.at[:, pl.ds((i - 2) * VT, VT)],
                        sems.at[s],
                    ).wait()

                acc[s] = lax.dot_general(
                    h_ref[...], w_ref[...],
                    (((1,), (0,)), ((), ())),
                    preferred_element_type=jnp.float32,
                ) + b_ref[...]

                cp = pltpu.make_async_copy(
                    acc.at[s],
                    o_hbm.at[:, pl.ds(i * VT, VT)],
                    sems.at[s],
                )
                cp.start()

                @pl.when(i == NFULL - 1)
                def _():
                    cp.wait()
                    pltpu.make_async_copy(
                        acc.at[1 - s],
                        o_hbm.at[:, pl.ds((i - 1) * VT, VT)],
                        sems.at[1 - s],
                    ).wait()

    return pl.pallas_call(
        body,
        grid=(NFULL,),
        compiler_params=pltpu.CompilerParams(
            dimension_semantics=("arbitrary",),
        ),
        in_specs=[
            pl.BlockSpec((B, D), lambda i: (0, 0)),
            pl.BlockSpec((D, VT), lambda i: (0, i)),
            pl.BlockSpec((1, VT), lambda i: (0, i)),
        ],
        out_specs=pl.BlockSpec(memory_space=pl.ANY),
        out_shape=jax.ShapeDtypeStruct((B, V), jnp.float32),
        scratch_shapes=[
            pltpu.VMEM((2, B, VT), jnp.float32),
            pltpu.SemaphoreType.DMA((2,)),
        ],
    )(h, W, b2)


TAILW = 1024
TAIL0 = (NFULL * VT) // TAILW


def _tc_logits_tail(h, W, b2, prev):
    def body(h_ref, w_ref, b_ref, _, o_ref):
        o_ref[...] = lax.dot_general(
            h_ref[...], w_ref[...],
            (((1,), (0,)), ((), ())),
            preferred_element_type=jnp.float32,
        ) + b_ref[...]

    ntail = (V - NFULL * VT + TAILW - 1) // TAILW
    return pl.pallas_call(
        body,
        grid=(ntail,),
        in_specs=[
            pl.BlockSpec((B, D), lambda i: (0, 0)),
            pl.BlockSpec((D, TAILW), lambda i: (0, TAIL0 + i)),
            pl.BlockSpec((1, TAILW), lambda i: (0, TAIL0 + i)),
            pl.BlockSpec(memory_space=pl.ANY),
        ],
        out_specs=pl.BlockSpec((B, TAILW), lambda i: (0, TAIL0 + i)),
        out_shape=jax.ShapeDtypeStruct((B, V), jnp.float32),
        input_output_aliases={3: 0},
    )(h, W, b2, prev)


def kernel(x, embed_table, W, b):
    table128 = _tc_prep_table(embed_table.T)
    h = _sc_embed_mean(x, table128)
    b2 = b.reshape(1, V)
    wt = W.T
    out = _tc_logits(h, wt, b2)
    return _tc_logits_tail(h, wt, b2, out)


# speedup vs baseline: 1.0706x; 1.0706x over previous
import functools

import jax
import jax.numpy as jnp
from jax import lax
from jax.experimental import pallas as pl
from jax.experimental.pallas import tpu as pltpu
from jax.experimental.pallas import tpu_sc as plsc

B = 1024
L = 50
D = 64
V = 119547

NC = 2
NS = 16
NW = NC * NS
BPW = B // NW

LANES = 16


CT = 4096
NPREP = (V + CT - 1) // CT
WAVE = BPW // 2


def _tc_prep_table(wt_table):
    def body(t_ref, o_ref):
        o_ref[:, 0:D] = lax.transpose(t_ref[...], (1, 0))

    return pl.pallas_call(
        body,
        grid=(NPREP,),
        in_specs=[pl.BlockSpec((D, CT), lambda i: (0, i))],
        out_specs=pl.BlockSpec((CT, 128), lambda i: (i, 0)),
        out_shape=jax.ShapeDtypeStruct((V, 128), jnp.float32),
    )(wt_table)


def _sc_embed_mean(x, table128):
    mesh = plsc.VectorSubcoreMesh(core_axis_name="c", subcore_axis_name="s")

    @functools.partial(
        pl.kernel,
        out_type=jax.ShapeDtypeStruct((B, D), jnp.float32),
        mesh=mesh,
        compiler_params=pltpu.CompilerParams(use_tc_tiling_on_sc=False),
        scratch_types=[
            pltpu.VMEM((BPW, L), jnp.int32),
            pltpu.VMEM((WAVE * L, 128), jnp.float32),
            pltpu.VMEM((BPW, D), jnp.float32),
            pltpu.SemaphoreType.DMA,
        ],
    )
    def k(x_hbm, table_hbm, out_hbm, idx_v, rows_v, h_v, sem):
        wid = lax.axis_index("s") * NC + lax.axis_index("c")
        base = wid * BPW
        pltpu.sync_copy(x_hbm.at[pl.ds(base, BPW)], idx_v)

        @pl.loop(0, BPW, step=WAVE)
        def _(w):
            @pl.loop(0, WAVE)
            def _(j):
                pltpu.async_copy(
                    table_hbm.at[idx_v.at[w + j]],
                    rows_v.at[pl.ds(j * L, L)], sem)

            @pl.loop(0, WAVE)
            def _(j):
                pltpu.make_async_copy(
                    table_hbm.at[idx_v.at[w + j]],
                    rows_v.at[pl.ds(j * L, L)], sem
                ).wait()

            @pl.loop(0, WAVE)
            def _(j):
                for c in range(0, D, LANES):
                    acc = rows_v[j * L, pl.ds(c, LANES)]
                    for l in range(1, L):
                        acc = acc + rows_v[j * L + l, pl.ds(c, LANES)]
                    h_v[w + j, pl.ds(c, LANES)] = acc * (1.0 / L)

        pltpu.sync_copy(h_v, out_hbm.at[pl.ds(base, BPW)])

    return k(x, table128)


VT = 4096
GRID_V = (V + VT - 1) // VT


NFULL = V // VT
REM = V - NFULL * VT


def _tc_logits(h, W, b2):
    def body(h_ref, w_ref, b_ref, o_hbm, acc, sems):
        i = pl.program_id(0)
        slot = lax.rem(i, 2)

        for s in (0, 1):
            @pl.when(slot == s)
            def _(s=s):
                @pl.when(i >= 2)
                def _():
                    pltpu.make_async_copy(
                        acc.at[s],
                        o_hbm.at[:, pl.ds((i - 2) * VT, VT)],
                        sems.at[s],
                    ).wait()

                acc[s] = lax.dot_general(
                    h_ref[...], w_ref[...],
                    (((1,), (0,)), ((), ())),
                    preferred_element_type=jnp.float32,
                ) + b_ref[...]

                cp = pltpu.make_async_copy(
                    acc.at[s],
                    o_hbm.at[:, pl.ds(i * VT, VT)],
                    sems.at[s],
                )
                cp.start()

                @pl.when(i == NFULL - 1)
                def _():
                    cp.wait()
                    pltpu.make_async_copy(
                        acc.at[1 - s],
                        o_hbm.at[:, pl.ds((i - 1) * VT, VT)],
                        sems.at[1 - s],
                    ).wait()

    return pl.pallas_call(
        body,
        grid=(NFULL,),
        compiler_params=pltpu.CompilerParams(
            dimension_semantics=("arbitrary",),
        ),
        in_specs=[
            pl.BlockSpec((B, D), lambda i: (0, 0)),
            pl.BlockSpec((D, VT), lambda i: (0, i)),
            pl.BlockSpec((1, VT), lambda i: (0, i)),
        ],
        out_specs=pl.BlockSpec(memory_space=pl.ANY),
        out_shape=jax.ShapeDtypeStruct((B, V), jnp.float32),
        scratch_shapes=[
            pltpu.VMEM((2, B, VT), jnp.float32),
            pltpu.SemaphoreType.DMA((2,)),
        ],
    )(h, W, b2)


TAILW = 1024
TAIL0 = (NFULL * VT) // TAILW


def _tc_logits_tail(h, W, b2, prev):
    def body(h_ref, w_ref, b_ref, _, o_ref):
        o_ref[...] = lax.dot_general(
            h_ref[...], w_ref[...],
            (((1,), (0,)), ((), ())),
            preferred_element_type=jnp.float32,
        ) + b_ref[...]

    ntail = (V - NFULL * VT + TAILW - 1) // TAILW
    return pl.pallas_call(
        body,
        grid=(ntail,),
        in_specs=[
            pl.BlockSpec((B, D), lambda i: (0, 0)),
            pl.BlockSpec((D, TAILW), lambda i: (0, TAIL0 + i)),
            pl.BlockSpec((1, TAILW), lambda i: (0, TAIL0 + i)),
            pl.BlockSpec(memory_space=pl.ANY),
        ],
        out_specs=pl.BlockSpec((B, TAILW), lambda i: (0, TAIL0 + i)),
        out_shape=jax.ShapeDtypeStruct((B, V), jnp.float32),
        input_output_aliases={3: 0},
    )(h, W, b2, prev)


def kernel(x, embed_table, W, b):
    table128 = _tc_prep_table(embed_table.T)
    h = _sc_embed_mean(x, table128)
    b2 = b.reshape(1, V)
    wt = W.T
    out = _tc_logits(h, wt, b2)
    return _tc_logits_tail(h, wt, b2, out)


# speedup vs baseline: 1.1077x; 1.0347x over previous
import functools

import jax
import jax.numpy as jnp
from jax import lax
from jax.experimental import pallas as pl
from jax.experimental.pallas import tpu as pltpu
from jax.experimental.pallas import tpu_sc as plsc

B = 1024
L = 50
D = 64
V = 119547

NC = 2
NS = 16
NW = NC * NS
BPW = B // NW

LANES = 16


CT = 8192
NPREP = (V + CT - 1) // CT
WAVE = BPW // 2


def _tc_prep_table(wt_table):
    def body(t_ref, o_ref):
        o_ref[:, 0:D] = lax.transpose(t_ref[...], (1, 0))

    return pl.pallas_call(
        body,
        grid=(NPREP,),
        in_specs=[pl.BlockSpec((D, CT), lambda i: (0, i))],
        out_specs=pl.BlockSpec((CT, 128), lambda i: (i, 0)),
        out_shape=jax.ShapeDtypeStruct((V, 128), jnp.float32),
    )(wt_table)


def _sc_embed_mean(x, table128):
    mesh = plsc.VectorSubcoreMesh(core_axis_name="c", subcore_axis_name="s")

    @functools.partial(
        pl.kernel,
        out_type=jax.ShapeDtypeStruct((B, D), jnp.float32),
        mesh=mesh,
        compiler_params=pltpu.CompilerParams(use_tc_tiling_on_sc=False),
        scratch_types=[
            pltpu.VMEM((BPW, L), jnp.int32),
            pltpu.VMEM((WAVE * L, 128), jnp.float32),
            pltpu.VMEM((BPW, D), jnp.float32),
            pltpu.SemaphoreType.DMA,
        ],
    )
    def k(x_hbm, table_hbm, out_hbm, idx_v, rows_v, h_v, sem):
        wid = lax.axis_index("s") * NC + lax.axis_index("c")
        base = wid * BPW
        pltpu.sync_copy(x_hbm.at[pl.ds(base, BPW)], idx_v)

        @pl.loop(0, BPW, step=WAVE)
        def _(w):
            @pl.loop(0, WAVE)
            def _(j):
                pltpu.async_copy(
                    table_hbm.at[idx_v.at[w + j]],
                    rows_v.at[pl.ds(j * L, L)], sem)

            @pl.loop(0, WAVE)
            def _(j):
                pltpu.make_async_copy(
                    table_hbm.at[idx_v.at[w + j]],
                    rows_v.at[pl.ds(j * L, L)], sem
                ).wait()

            @pl.loop(0, WAVE)
            def _(j):
                for c in range(0, D, LANES):
                    acc = rows_v[j * L, pl.ds(c, LANES)]
                    for l in range(1, L):
                        acc = acc + rows_v[j * L + l, pl.ds(c, LANES)]
                    h_v[w + j, pl.ds(c, LANES)] = acc * (1.0 / L)

        pltpu.sync_copy(h_v, out_hbm.at[pl.ds(base, BPW)])

    return k(x, table128)


VT = 4096
GRID_V = (V + VT - 1) // VT


NFULL = V // VT
REM = V - NFULL * VT


def _tc_logits(h, W, b2):
    def body(h_ref, w_ref, b_ref, o_hbm, acc, sems):
        i = pl.program_id(0)
        slot = lax.rem(i, 2)

        for s in (0, 1):
            @pl.when(slot == s)
            def _(s=s):
                @pl.when(i >= 2)
                def _():
                    pltpu.make_async_copy(
                        acc.at[s],
                        o_hbm.at[:, pl.ds((i - 2) * VT, VT)],
                        sems.at[s],
                    ).wait()

                acc[s] = lax.dot_general(
                    h_ref[...], w_ref[...],
                    (((1,), (0,)), ((), ())),
                    preferred_element_type=jnp.float32,
                ) + b_ref[...]

                cp = pltpu.make_async_copy(
                    acc.at[s],
                    o_hbm.at[:, pl.ds(i * VT, VT)],
                    sems.at[s],
                )
                cp.start()

                @pl.when(i == NFULL - 1)
                def _():
                    cp.wait()
                    pltpu.make_async_copy(
                        acc.at[1 - s],
                        o_hbm.at[:, pl.ds((i - 1) * VT, VT)],
                        sems.at[1 - s],
                    ).wait()

    return pl.pallas_call(
        body,
        grid=(NFULL,),
        compiler_params=pltpu.CompilerParams(
            dimension_semantics=("arbitrary",),
        ),
        in_specs=[
            pl.BlockSpec((B, D), lambda i: (0, 0)),
            pl.BlockSpec((D, VT), lambda i: (0, i)),
            pl.BlockSpec((1, VT), lambda i: (0, i)),
        ],
        out_specs=pl.BlockSpec(memory_space=pl.ANY),
        out_shape=jax.ShapeDtypeStruct((B, V), jnp.float32),
        scratch_shapes=[
            pltpu.VMEM((2, B, VT), jnp.float32),
            pltpu.SemaphoreType.DMA((2,)),
        ],
    )(h, W, b2)


TAILW = 1024
TAIL0 = (NFULL * VT) // TAILW


def _tc_logits_tail(h, W, b2, prev):
    def body(h_ref, w_ref, b_ref, _, o_ref):
        o_ref[...] = lax.dot_general(
            h_ref[...], w_ref[...],
            (((1,), (0,)), ((), ())),
            preferred_element_type=jnp.float32,
        ) + b_ref[...]

    ntail = (V - NFULL * VT + TAILW - 1) // TAILW
    return pl.pallas_call(
        body,
        grid=(ntail,),
        in_specs=[
            pl.BlockSpec((B, D), lambda i: (0, 0)),
            pl.BlockSpec((D, TAILW), lambda i: (0, TAIL0 + i)),
            pl.BlockSpec((1, TAILW), lambda i: (0, TAIL0 + i)),
            pl.BlockSpec(memory_space=pl.ANY),
        ],
        out_specs=pl.BlockSpec((B, TAILW), lambda i: (0, TAIL0 + i)),
        out_shape=jax.ShapeDtypeStruct((B, V), jnp.float32),
        input_output_aliases={3: 0},
    )(h, W, b2, prev)


def kernel(x, embed_table, W, b):
    table128 = _tc_prep_table(embed_table.T)
    h = _sc_embed_mean(x, table128)
    b2 = b.reshape(1, V)
    wt = W.T
    out = _tc_logits(h, wt, b2)
    return _tc_logits_tail(h, wt, b2, out)
